# Initial kernel scaffold; baseline (speedup 1.0000x reference)
import jax
import jax.numpy as jnp
from jax.experimental import pallas as pl


def kernel(x, W1, b1, W2, b2, emb, Wc, bc, task):
    raise NotImplementedError("write your pallas kernel here")


# speedup vs baseline: 2.7195x; 2.7195x over previous
import jax
import jax.numpy as jnp
from jax.experimental import pallas as pl

E = 10
B = 16
CIN = 96
COUT = 96
H = 28
W = 28
HO = H - 2
WO = W - 2
NPIX = H * W
NPAD = 792
NROW = HO * W


def _moe_kernel(x_ref, w1_ref, b1_ref, w2_ref, b2_ref, emb_ref, wt_ref,
                bc_ref, out_ref):
    xb = x_ref[0]
    r = jnp.sum(xb[:NPIX, :], axis=0, keepdims=True) * (1.0 / NPIX)
    h1 = jnp.maximum(
        jnp.dot(r, w1_ref[:, :], preferred_element_type=jnp.float32)
        + b1_ref[:, :], 0.0)
    r2 = (jnp.dot(h1, w2_ref[:, :], preferred_element_type=jnp.float32)
          + b2_ref[:, :])
    embv = emb_ref[:, :]
    d = jax.lax.dot_general(r2, embv, (((1,), (1,)), ((), ())),
                            preferred_element_type=jnp.float32)
    ssum = jax.lax.dot_general(
        jnp.ones((1, 64), jnp.float32), embv * embv,
        (((1,), (1,)), ((), ())),
        preferred_element_type=jnp.float32)
    sim = d / (jnp.sqrt(ssum) + 1e-8)
    e_idx = jnp.argmax(sim)

    acc = bc_ref[e_idx]
    acc = jnp.broadcast_to(acc, (NROW, COUT))
    for di in range(3):
        for dj in range(3):
            base = di * W + dj
            fs = x_ref[0, base:base + NROW, :]
            wk = wt_ref[e_idx, di, dj]
            acc = acc + jnp.dot(fs, wk, preferred_element_type=jnp.float32)
    a3 = acc.reshape(HO, W, COUT)
    out_ref[0] = a3[:, :WO, :]


def kernel(x, W1, b1, W2, b2, emb, Wc, bc, task):
    xt = jnp.transpose(x, (0, 2, 3, 1)).reshape(B, NPIX, CIN)
    xf = jnp.pad(xt, ((0, 0), (0, NPAD - NPIX), (0, 0)))
    wt = jnp.transpose(Wc, (0, 3, 4, 2, 1))
    b1r = b1.reshape(1, 128)
    b2r = b2.reshape(1, 64)
    bcr = bc.reshape(E, 1, COUT)

    out = pl.pallas_call(
        _moe_kernel,
        grid=(B,),
        in_specs=[
            pl.BlockSpec((1, NPAD, CIN), lambda b: (b, 0, 0)),
            pl.BlockSpec((CIN, 128), lambda b: (0, 0)),
            pl.BlockSpec((1, 128), lambda b: (0, 0)),
            pl.BlockSpec((128, 64), lambda b: (0, 0)),
            pl.BlockSpec((1, 64), lambda b: (0, 0)),
            pl.BlockSpec((E, 64), lambda b: (0, 0)),
            pl.BlockSpec((E, 3, 3, CIN, COUT), lambda b: (0, 0, 0, 0, 0)),
            pl.BlockSpec((E, 1, COUT), lambda b: (0, 0, 0)),
        ],
        out_specs=pl.BlockSpec((1, HO, WO, COUT), lambda b: (b, 0, 0, 0)),
        out_shape=jax.ShapeDtypeStruct((B, HO, WO, COUT), jnp.float32),
    )(xf, W1.T, b1r, W2.T, b2r, emb, wt, bcr)

    return jnp.transpose(out, (0, 3, 1, 2))
